# baseline (speedup 1.0000x reference)
import jax
import jax.numpy as jnp
from jax import lax
from jax.experimental import pallas as pl
from jax.experimental.pallas import tpu as pltpu
from jax.experimental.pallas import tpu_sc as plsc

D_MODEL = 1024
BATCH = 128
L = 16

NC = 2
NS = 16
NW = NC * NS
BPW = BATCH // NW


def _body(tok_hbm, pidx_hbm, table_hbm, pos_hbm, out_hbm,
          idx_v, pidx_v, rows_v, pos_v, sem_t, sem_p):
    c = lax.axis_index("c")
    s = lax.axis_index("s")
    wid = s * NC + c

    pltpu.sync_copy(tok_hbm.at[wid], idx_v)
    pltpu.sync_copy(pidx_hbm, pidx_v)

    cp_t = pltpu.async_copy(table_hbm.at[idx_v], rows_v, sem_t)
    cp_p = pltpu.async_copy(pos_hbm.at[pidx_v], pos_v, sem_p)
    cp_p.wait()
    cp_t.wait()

    def add_chunk(j, carry):
        off = j * L
        pc = pos_v[0, pl.ds(off, L)]
        for b in range(BPW):
            rows_v[b, pl.ds(off, L)] += pc
        return carry

    lax.fori_loop(0, D_MODEL // L, add_chunk, 0)

    pltpu.sync_copy(rows_v, out_hbm.at[pl.ds(wid * BPW, BPW)])


def kernel(next_token, idx, speech_emb, pos_emb):
    tok2d = next_token.reshape(NW, BPW).astype(jnp.int32)
    posidx = (idx + 1).astype(jnp.int32)
    mesh = plsc.VectorSubcoreMesh(
        core_axis_name="c", subcore_axis_name="s",
        num_cores=NC, num_subcores=NS)
    out = pl.kernel(
        _body,
        mesh=mesh,
        out_type=jax.ShapeDtypeStruct((BATCH, D_MODEL), jnp.float32),
        scratch_types=[
            pltpu.VMEM((BPW,), jnp.int32),
            pltpu.VMEM((1,), jnp.int32),
            pltpu.VMEM((BPW, D_MODEL), jnp.float32),
            pltpu.VMEM((1, D_MODEL), jnp.float32),
            pltpu.SemaphoreType.DMA,
            pltpu.SemaphoreType.DMA,
        ],
        name="speech_embedding_sc",
    )(tok2d, posidx, speech_emb, pos_emb)
    return out.reshape(BATCH, 1, D_MODEL)
